# baseline (speedup 1.0000x reference)
import functools

import jax
import jax.numpy as jnp
from jax import lax
from jax.experimental import pallas as pl
from jax.experimental.pallas import tpu as pltpu
from jax.experimental.pallas import tpu_sc as plsc

_LANES = 16


@functools.partial(jax.jit, static_argnames=("nw",))
def _sc_embed(x, token_table, pos_table, *, nw):
    batch, seq = x.shape
    hidden = token_table.shape[1]
    spb = 128
    nsb = seq // spb
    bpw = batch * nsb // nw
    lanes = hidden // _LANES

    mesh = plsc.VectorSubcoreMesh(core_axis_name="c", subcore_axis_name="s")

    @functools.partial(
        pl.kernel,
        out_type=jax.ShapeDtypeStruct((batch * seq, hidden), jnp.float32),
        mesh=mesh,
        scratch_types=[
            pltpu.VMEM((bpw, spb), jnp.int32),
            pltpu.VMEM((bpw * spb, hidden), jnp.float32),
            pltpu.VMEM((spb, hidden), jnp.float32),
            [pltpu.SemaphoreType.DMA] * bpw,
            [pltpu.SemaphoreType.DMA] * bpw,
            pltpu.SemaphoreType.DMA,
            pltpu.SemaphoreType.DMA,
        ],
    )
    def body(x_hbm, tok_hbm, pos_hbm, out_hbm, idx_v, rows_v, pos_v,
             isems, gsems, psem, wsem):
        wid = lax.axis_index("s") * 2 + lax.axis_index("c")
        sb = wid // (batch // bpw)
        b0 = (wid % (batch // bpw)) * bpw
        s0 = sb * spb

        icps = [
            pltpu.async_copy(
                x_hbm.at[pl.ds(b0 + b, 1), pl.ds(s0, spb)],
                idx_v.at[pl.ds(b, 1)],
                isems[b],
            )
            for b in range(bpw)
        ]
        pcp = pltpu.async_copy(pos_hbm.at[pl.ds(s0, spb)], pos_v, psem)

        gcps = []
        for b in range(bpw):
            icps[b].wait()
            gcps.append(
                pltpu.async_copy(
                    tok_hbm.at[idx_v.at[b]],
                    rows_v.at[pl.ds(b * spb, spb)],
                    gsems[b],
                )
            )
        pcp.wait()

        wcps = []
        for b in range(bpw):
            gcps[b].wait()

            @plsc.parallel_loop(0, spb, unroll=4)
            def _(r, _b=b):
                for j in range(lanes):
                    sl = pl.ds(j * _LANES, _LANES)
                    plsc.addupdate(rows_v.at[_b * spb + r, sl], pos_v[r, sl])

            wcps.append(
                pltpu.async_copy(
                    rows_v.at[pl.ds(b * spb, spb)],
                    out_hbm.at[pl.ds((b0 + b) * seq + s0, spb)],
                    wsem,
                )
            )
        for cp in wcps:
            cp.wait()

    return body(x, token_table, pos_table)


def kernel(x, token_table, pos_table):
    batch, seq = x.shape
    hidden = token_table.shape[1]
    out = _sc_embed(x.astype(jnp.int32), token_table, pos_table, nw=32)
    return out.reshape(batch, seq, hidden)


# speedup vs baseline: 1.0203x; 1.0203x over previous
import functools

import jax
import jax.numpy as jnp
from jax import lax
from jax.experimental import pallas as pl
from jax.experimental.pallas import tpu as pltpu
from jax.experimental.pallas import tpu_sc as plsc

_LANES = 16


@functools.partial(jax.jit, static_argnames=("nw",))
def _sc_embed(x, token_table, pos_table, *, nw):
    batch, seq = x.shape
    hidden = token_table.shape[1]
    spw = seq // nw
    win = 128
    lanes = hidden // _LANES

    mesh = plsc.VectorSubcoreMesh(core_axis_name="c", subcore_axis_name="s")

    @functools.partial(
        pl.kernel,
        out_type=jax.ShapeDtypeStruct((batch * seq, hidden), jnp.float32),
        mesh=mesh,
        scratch_types=[
            pltpu.VMEM((batch, win), jnp.int32),
            pltpu.VMEM((batch * spw, hidden), jnp.float32),
            pltpu.VMEM((spw, hidden), jnp.float32),
            [pltpu.SemaphoreType.DMA] * 4,
            [pltpu.SemaphoreType.DMA] * 4,
            pltpu.SemaphoreType.DMA,
            pltpu.SemaphoreType.DMA,
        ],
    )
    def body(x_hbm, tok_hbm, pos_hbm, out_hbm, idx_v, rows_v, pos_v,
             isems, gsems, psem, wsem):
        wid = lax.axis_index("s") * 2 + lax.axis_index("c")
        s0 = wid * spw
        w0 = (s0 // win) * win
        off = s0 - w0

        icps = [
            pltpu.async_copy(
                x_hbm.at[pl.ds(b, 1), pl.ds(w0, win)],
                idx_v.at[pl.ds(b, 1)],
                isems[b],
            )
            for b in range(batch)
        ]
        pcp = pltpu.async_copy(pos_hbm.at[pl.ds(s0, spw)], pos_v, psem)

        gcps = []
        for b in range(batch):
            icps[b].wait()
            gcps.append(
                pltpu.async_copy(
                    tok_hbm.at[idx_v.at[b, pl.ds(off, spw)]],
                    rows_v.at[pl.ds(b * spw, spw)],
                    gsems[b],
                )
            )
        pcp.wait()

        wcps = []
        for b in range(batch):
            gcps[b].wait()

            @plsc.parallel_loop(0, spw, unroll=2)
            def _(r, _b=b):
                for j in range(lanes):
                    sl = pl.ds(j * _LANES, _LANES)
                    plsc.addupdate(rows_v.at[_b * spw + r, sl], pos_v[r, sl])

            wcps.append(
                pltpu.async_copy(
                    rows_v.at[pl.ds(b * spw, spw)],
                    out_hbm.at[pl.ds(b * seq + s0, spw)],
                    wsem,
                )
            )
        for cp in wcps:
            cp.wait()

    return body(x, token_table, pos_table)


def kernel(x, token_table, pos_table):
    batch, seq = x.shape
    hidden = token_table.shape[1]
    out = _sc_embed(x.astype(jnp.int32), token_table, pos_table, nw=32)
    return out.reshape(batch, seq, hidden)
